# Initial kernel scaffold; baseline (speedup 1.0000x reference)
import jax
import jax.numpy as jnp
from jax.experimental import pallas as pl


def kernel(x, edge_index, edge_attr, batch, embed, W1, b1, W2, b2, W3, b3, Wlin, blin, Wih0, Whh0, bih0, bhh0, Wih1, Whh1, bih1, bhh1, Wd, bd):
    raise NotImplementedError("write your pallas kernel here")


# speedup vs baseline: 13.9410x; 13.9410x over previous
import functools

import jax
import jax.numpy as jnp
from jax import lax
from jax.experimental import pallas as pl
from jax.experimental.pallas import tpu as pltpu
from jax.experimental.pallas import tpu_sc as plsc

N = 10000
E = 320000
EMB = 128
HID = 128
NG = 64
SEQ = 20
LS = 128
NV = 5000

NC = 2
NS = 16
NW = NC * NS
K = 80
CPW = E // (NW * K)
RPT = 624
TAIL_BASE = NS * RPT
TAIL = N - TAIL_BASE
ECH = N // K

RB = 1000
GR = N // RB

_MESH = plsc.VectorSubcoreMesh(core_axis_name="c", subcore_axis_name="s")


def _stripe_copy(sid, src, dst):
    pltpu.sync_copy(src.at[pl.ds(sid * RPT, RPT)],
                    dst.at[pl.ds(sid * RPT, RPT)])

    @pl.when(sid == NS - 1)
    def _():
        pltpu.sync_copy(src.at[pl.ds(TAIL_BASE, TAIL)],
                        dst.at[pl.ds(TAIL_BASE, TAIL)])


@functools.partial(
    pl.kernel,
    out_type=(
        jax.ShapeDtypeStruct((N, EMB), jnp.float32),
        jax.ShapeDtypeStruct((NC, N, HID), jnp.float32),
    ),
    mesh=_MESH,
    scratch_types=[
        pltpu.VMEM((K,), jnp.int32),
        pltpu.VMEM((K, EMB), jnp.float32),
        pltpu.VMEM((CPW, K), jnp.int32),
        pltpu.VMEM((K, HID), jnp.float32),
        pltpu.SemaphoreType.DMA,
        pltpu.VMEM_SHARED((N, HID), jnp.float32),
    ],
)
def _embed_deg(emb_hbm, xidx_hbm, dst_hbm, ones_hbm, z_hbm,
               h0_hbm, degp_hbm, xbuf, rows, didx, ones_v, sem, deg_sh):
    cid = lax.axis_index("c")
    sid = lax.axis_index("s")
    wid = sid * NC + cid

    _stripe_copy(sid, z_hbm, deg_sh)
    pltpu.sync_copy(ones_hbm, ones_v)
    pltpu.sync_copy(dst_hbm.at[wid], didx)

    for j in range((ECH + NW - 1) // NW):
        ch = wid + j * NW

        @pl.when(ch < ECH)
        def _():
            pltpu.sync_copy(xidx_hbm.at[pl.ds(ch * K, K)], xbuf)
            pltpu.async_copy(emb_hbm.at[xbuf], rows, sem).wait()
            pltpu.sync_copy(rows, h0_hbm.at[pl.ds(ch * K, K)])

    plsc.subcore_barrier()

    def body(i, carry):
        pltpu.sync_copy(ones_v, deg_sh.at[didx.at[i]], add=True)
        return carry

    lax.fori_loop(0, CPW, body, 0)
    plsc.subcore_barrier()
    _stripe_copy(sid, deg_sh, degp_hbm.at[cid])


@functools.partial(
    pl.kernel,
    out_type=jax.ShapeDtypeStruct((NC, N, HID), jnp.float32),
    mesh=_MESH,
    scratch_types=[
        pltpu.VMEM((CPW, K), jnp.int32),
        pltpu.VMEM((CPW, K), jnp.int32),
        pltpu.VMEM((K, HID), jnp.float32),
        pltpu.SemaphoreType.DMA,
        pltpu.VMEM_SHARED((N, HID), jnp.float32),
    ],
)
def _segsum(g_hbm, src_hbm, dst_hbm, z_hbm,
            part_hbm, sidx, didx, rows, sem, acc_sh):
    cid = lax.axis_index("c")
    sid = lax.axis_index("s")
    wid = sid * NC + cid

    _stripe_copy(sid, z_hbm, acc_sh)
    pltpu.sync_copy(src_hbm.at[wid], sidx)
    pltpu.sync_copy(dst_hbm.at[wid], didx)
    plsc.subcore_barrier()

    def body(i, carry):
        pltpu.async_copy(g_hbm.at[sidx.at[i]], rows, sem).wait()
        pltpu.sync_copy(rows, acc_sh.at[didx.at[i]], add=True)
        return carry

    lax.fori_loop(0, CPW, body, 0)
    plsc.subcore_barrier()
    _stripe_copy(sid, acc_sh, part_hbm.at[cid])


def _t1_body(degp_ref, h0_ref, w_ref, dinv_ref, g_ref):
    d = degp_ref[0] + degp_ref[1]
    deg = d[:, 0:1] + 1.0
    dinv = jnp.broadcast_to(lax.rsqrt(deg), (RB, HID))
    dinv_ref[...] = dinv
    g_ref[...] = jnp.dot(h0_ref[...] * dinv, w_ref[...],
                         preferred_element_type=jnp.float32)


def _t1(degp, h0, W1):
    return pl.pallas_call(
        _t1_body,
        grid=(GR,),
        in_specs=[
            pl.BlockSpec((NC, RB, HID), lambda i: (0, i, 0)),
            pl.BlockSpec((RB, EMB), lambda i: (i, 0)),
            pl.BlockSpec((EMB, HID), lambda i: (0, 0)),
        ],
        out_specs=[
            pl.BlockSpec((RB, HID), lambda i: (i, 0)),
            pl.BlockSpec((RB, HID), lambda i: (i, 0)),
        ],
        out_shape=[
            jax.ShapeDtypeStruct((N, HID), jnp.float32),
            jax.ShapeDtypeStruct((N, HID), jnp.float32),
        ],
    )(degp, h0, W1)


def _t23_body(part_ref, g_ref, dinv_ref, b_ref, w_ref, gn_ref):
    dinv = dinv_ref[...]
    h = dinv * (part_ref[0] + part_ref[1] + g_ref[...]) + b_ref[...]
    h = jnp.maximum(h, 0.0)
    gn_ref[...] = jnp.dot(h * dinv, w_ref[...],
                          preferred_element_type=jnp.float32)


def _t23(part, g, dinv, b, W):
    return pl.pallas_call(
        _t23_body,
        grid=(GR,),
        in_specs=[
            pl.BlockSpec((NC, RB, HID), lambda i: (0, i, 0)),
            pl.BlockSpec((RB, HID), lambda i: (i, 0)),
            pl.BlockSpec((RB, HID), lambda i: (i, 0)),
            pl.BlockSpec((1, HID), lambda i: (0, 0)),
            pl.BlockSpec((HID, HID), lambda i: (0, 0)),
        ],
        out_specs=pl.BlockSpec((RB, HID), lambda i: (i, 0)),
        out_shape=jax.ShapeDtypeStruct((N, HID), jnp.float32),
    )(part, g, dinv, b, W)


def _t4a_body(part_ref, g_ref, dinv_ref, b_ref, batch_ref,
              wlin_ref, blin_ref, wih0_ref, whh0_ref, bb0_ref,
              wih1_ref, whh1_ref, bb1_ref, o_ref, sum_scr, cnt_scr):
    i = pl.program_id(0)
    dinv = dinv_ref[...]
    h = dinv * (part_ref[0] + part_ref[1] + g_ref[...]) + b_ref[...]
    b_blk = batch_ref[0, 0, :]
    gids = lax.broadcasted_iota(jnp.int32, (NG, RB), 0)
    M = jnp.where(gids == jnp.broadcast_to(b_blk[None, :], (NG, RB)),
                  1.0, 0.0)
    psum = jnp.dot(M, h, preferred_element_type=jnp.float32)
    pcnt = jnp.broadcast_to(jnp.sum(M, axis=1, keepdims=True), (NG, HID))

    @pl.when(i == 0)
    def _():
        sum_scr[...] = psum
        cnt_scr[...] = pcnt

    @pl.when(i > 0)
    def _():
        sum_scr[...] += psum
        cnt_scr[...] += pcnt

    @pl.when(i == GR - 1)
    def _():
        pooled = sum_scr[...] / jnp.maximum(cnt_scr[...], 1.0)
        lin = jnp.dot(pooled, wlin_ref[...],
                      preferred_element_type=jnp.float32) + blin_ref[...]
        xw0 = jnp.dot(lin, wih0_ref[...],
                      preferred_element_type=jnp.float32) + bb0_ref[...]
        hh = jnp.zeros((NG, LS), jnp.float32)
        cc = jnp.zeros((NG, LS), jnp.float32)
        ys = []
        for t in range(SEQ):
            gts = xw0 + jnp.dot(hh, whh0_ref[...],
                                preferred_element_type=jnp.float32)
            ig = jax.nn.sigmoid(gts[:, 0:LS])
            fg = jax.nn.sigmoid(gts[:, LS:2 * LS])
            gg = jnp.tanh(gts[:, 2 * LS:3 * LS])
            og = jax.nn.sigmoid(gts[:, 3 * LS:4 * LS])
            cc = fg * cc + ig * gg
            hh = og * jnp.tanh(cc)
            ys.append(hh)
        hh1 = jnp.zeros((NG, LS), jnp.float32)
        cc1 = jnp.zeros((NG, LS), jnp.float32)
        for t in range(SEQ):
            gts = (jnp.dot(ys[t], wih1_ref[...],
                           preferred_element_type=jnp.float32)
                   + jnp.dot(hh1, whh1_ref[...],
                             preferred_element_type=jnp.float32)
                   + bb1_ref[...])
            ig = jax.nn.sigmoid(gts[:, 0:LS])
            fg = jax.nn.sigmoid(gts[:, LS:2 * LS])
            gg = jnp.tanh(gts[:, 2 * LS:3 * LS])
            og = jax.nn.sigmoid(gts[:, 3 * LS:4 * LS])
            cc1 = fg * cc1 + ig * gg
            hh1 = og * jnp.tanh(cc1)
            o_ref[t] = hh1


def _t4a(part, g, dinv, b, batch3, Wlin, blin, Wih0T, Whh0T, bb0,
         Wih1T, Whh1T, bb1):
    full = lambda shape: pl.BlockSpec(shape, lambda i: tuple(0 for _ in shape))
    return pl.pallas_call(
        _t4a_body,
        grid=(GR,),
        in_specs=[
            pl.BlockSpec((NC, RB, HID), lambda i: (0, i, 0)),
            pl.BlockSpec((RB, HID), lambda i: (i, 0)),
            pl.BlockSpec((RB, HID), lambda i: (i, 0)),
            full((1, HID)),
            pl.BlockSpec((1, 1, RB), lambda i: (i, 0, 0)),
            full((HID, HID)),
            full((1, HID)),
            full((HID, 4 * LS)),
            full((LS, 4 * LS)),
            full((1, 4 * LS)),
            full((LS, 4 * LS)),
            full((LS, 4 * LS)),
            full((1, 4 * LS)),
        ],
        out_specs=pl.BlockSpec((SEQ, NG, LS), lambda i: (0, 0, 0)),
        out_shape=jax.ShapeDtypeStruct((SEQ, NG, LS), jnp.float32),
        scratch_shapes=[
            pltpu.VMEM((NG, HID), jnp.float32),
            pltpu.VMEM((NG, HID), jnp.float32),
        ],
    )(part, g, dinv, b, batch3, Wlin, blin, Wih0T, Whh0T, bb0,
      Wih1T, Whh1T, bb1)


def _t4b_body(o_ref, wd_ref, bd_ref, out_ref):
    out_ref[...] = jnp.dot(o_ref[...], wd_ref[...],
                           preferred_element_type=jnp.float32) + bd_ref[...]


RB4 = 256


def _t4b(o2d, Wd, bd):
    return pl.pallas_call(
        _t4b_body,
        grid=(NG * SEQ // RB4,),
        in_specs=[
            pl.BlockSpec((RB4, LS), lambda j: (j, 0)),
            pl.BlockSpec((LS, NV), lambda j: (0, 0)),
            pl.BlockSpec((1, NV), lambda j: (0, 0)),
        ],
        out_specs=pl.BlockSpec((RB4, NV), lambda j: (j, 0)),
        out_shape=jax.ShapeDtypeStruct((NG * SEQ, NV), jnp.float32),
    )(o2d, Wd, bd)


def kernel(x, edge_index, edge_attr, batch, embed,
           W1, b1, W2, b2, W3, b3, Wlin, blin,
           Wih0, Whh0, bih0, bhh0, Wih1, Whh1, bih1, bhh1, Wd, bd):
    del edge_attr
    xidx = x[:, 0]
    src2 = edge_index[0].reshape(NW, CPW, K)
    dst2 = edge_index[1].reshape(NW, CPW, K)
    ones128 = jnp.ones((K, HID), jnp.float32)
    z128 = jnp.zeros((N, HID), jnp.float32)

    h0, degp = _embed_deg(embed, xidx, dst2, ones128, z128)
    dinv, g1 = _t1(degp, h0, W1)
    p1 = _segsum(g1, src2, dst2, z128)
    g2 = _t23(p1, g1, dinv, b1.reshape(1, HID), W2)
    p2 = _segsum(g2, src2, dst2, z128)
    g3 = _t23(p2, g2, dinv, b2.reshape(1, HID), W3)
    p3 = _segsum(g3, src2, dst2, z128)

    batch3 = batch.reshape(GR, 1, RB)
    o = _t4a(p3, g3, dinv, b3.reshape(1, HID), batch3,
             Wlin, blin.reshape(1, HID),
             Wih0.T, Whh0.T, (bih0 + bhh0).reshape(1, 4 * LS),
             Wih1.T, Whh1.T, (bih1 + bhh1).reshape(1, 4 * LS))
    o2d = jnp.swapaxes(o, 0, 1).reshape(NG * SEQ, LS)
    logits = _t4b(o2d, Wd, bd.reshape(1, NV))
    return logits.reshape(NG, SEQ, NV)


# speedup vs baseline: 22.5271x; 1.6159x over previous
import functools

import jax
import jax.numpy as jnp
from jax import lax
from jax.experimental import pallas as pl
from jax.experimental.pallas import tpu as pltpu
from jax.experimental.pallas import tpu_sc as plsc

N = 10000
E = 320000
EMB = 128
HID = 128
NG = 64
SEQ = 20
LS = 128
NV = 5000

NC = 2
NS = 16
NW = NC * NS
K = 80
CPW = E // (NW * K)
KS = 80
NCH = E // (NW * KS)
NBUF = 3
EPW = E // NW
RPT = 624
TAIL_BASE = NS * RPT
TAIL = N - TAIL_BASE
ECH = N // K

RB = 1000
GR = N // RB

_MESH = plsc.VectorSubcoreMesh(core_axis_name="c", subcore_axis_name="s")


def _stripe_copy(sid, src, dst):
    pltpu.sync_copy(src.at[pl.ds(sid * RPT, RPT)],
                    dst.at[pl.ds(sid * RPT, RPT)])

    @pl.when(sid == NS - 1)
    def _():
        pltpu.sync_copy(src.at[pl.ds(TAIL_BASE, TAIL)],
                        dst.at[pl.ds(TAIL_BASE, TAIL)])


@functools.partial(
    pl.kernel,
    out_type=(
        jax.ShapeDtypeStruct((N, EMB), jnp.float32),
        jax.ShapeDtypeStruct((NC, N, HID), jnp.float32),
    ),
    mesh=_MESH,
    scratch_types=[
        pltpu.VMEM((K,), jnp.int32),
        pltpu.VMEM((K, EMB), jnp.float32),
        pltpu.VMEM((CPW, K), jnp.int32),
        pltpu.VMEM((K, HID), jnp.float32),
        pltpu.SemaphoreType.DMA,
        pltpu.VMEM_SHARED((N, HID), jnp.float32),
    ],
)
def _embed_deg(emb_hbm, xidx_hbm, dst_hbm, ones_hbm, z_hbm,
               h0_hbm, degp_hbm, xbuf, rows, didx, ones_v, sem, deg_sh):
    cid = lax.axis_index("c")
    sid = lax.axis_index("s")
    wid = sid * NC + cid

    _stripe_copy(sid, z_hbm, deg_sh)
    pltpu.sync_copy(ones_hbm, ones_v)
    pltpu.sync_copy(dst_hbm.at[wid], didx)

    for j in range((ECH + NW - 1) // NW):
        ch = wid + j * NW

        @pl.when(ch < ECH)
        def _():
            pltpu.sync_copy(xidx_hbm.at[pl.ds(ch * K, K)], xbuf)
            pltpu.async_copy(emb_hbm.at[xbuf], rows, sem).wait()
            pltpu.sync_copy(rows, h0_hbm.at[pl.ds(ch * K, K)])

    plsc.subcore_barrier()

    def body(i, carry):
        pltpu.sync_copy(ones_v, deg_sh.at[didx.at[i]], add=True)
        return carry

    lax.fori_loop(0, CPW, body, 0)
    plsc.subcore_barrier()
    _stripe_copy(sid, deg_sh, degp_hbm.at[cid])


@functools.partial(
    pl.kernel,
    out_type=jax.ShapeDtypeStruct((NC, N, HID), jnp.float32),
    mesh=_MESH,
    scratch_types=[
        pltpu.VMEM((NCH, KS), jnp.int32),
        [pltpu.VMEM((1, KS), jnp.int32) for _ in range(NBUF)],
        [pltpu.VMEM((KS, HID), jnp.float32) for _ in range(NBUF)],
        [pltpu.SemaphoreType.DMA for _ in range(NBUF)],
        [pltpu.SemaphoreType.DMA for _ in range(NBUF)],
        pltpu.VMEM_SHARED((N, HID), jnp.float32),
    ],
)
def _segsum(g_hbm, src_hbm, dst_hbm, z_hbm,
            part_hbm, sidx, didx, rows, gsems, dsems, acc_sh):
    cid = lax.axis_index("c")
    sid = lax.axis_index("s")
    wid = sid * NC + cid

    _stripe_copy(sid, z_hbm, acc_sh)
    pltpu.sync_copy(src_hbm.at[wid], sidx)

    def fetch(i, b):
        pltpu.async_copy(dst_hbm.at[wid, i], didx[b], dsems[b])
        pltpu.async_copy(g_hbm.at[sidx.at[i]], rows[b], gsems[b])

    def drain(i, b):
        pltpu.make_async_copy(dst_hbm.at[wid, i], didx[b],
                              dsems[b]).wait()
        pltpu.make_async_copy(g_hbm.at[sidx.at[i]], rows[b],
                              gsems[b]).wait()
        pltpu.sync_copy(rows[b], acc_sh.at[didx[b].at[0]], add=True)

    for b in range(NBUF):
        fetch(b, b)
    plsc.subcore_barrier()

    def body(j, carry):
        for b in range(NBUF):
            i = NBUF * j + b
            drain(i, b)
            nxt = i + NBUF

            @pl.when(nxt < NCH)
            def _():
                fetch(nxt, b)
        return carry

    lax.fori_loop(0, NCH // NBUF, body, 0)
    for i in range(NBUF * (NCH // NBUF), NCH):
        drain(i, i % NBUF)
    plsc.subcore_barrier()
    _stripe_copy(sid, acc_sh, part_hbm.at[cid])


def _t1_body(degp_ref, h0_ref, w_ref, dinv_ref, g_ref):
    d = degp_ref[0] + degp_ref[1]
    deg = d[:, 0:1] + 1.0
    dinv = jnp.broadcast_to(lax.rsqrt(deg), (RB, HID))
    dinv_ref[...] = dinv
    g_ref[...] = jnp.dot(h0_ref[...] * dinv, w_ref[...],
                         preferred_element_type=jnp.float32)


def _t1(degp, h0, W1):
    return pl.pallas_call(
        _t1_body,
        grid=(GR,),
        in_specs=[
            pl.BlockSpec((NC, RB, HID), lambda i: (0, i, 0)),
            pl.BlockSpec((RB, EMB), lambda i: (i, 0)),
            pl.BlockSpec((EMB, HID), lambda i: (0, 0)),
        ],
        out_specs=[
            pl.BlockSpec((RB, HID), lambda i: (i, 0)),
            pl.BlockSpec((RB, HID), lambda i: (i, 0)),
        ],
        out_shape=[
            jax.ShapeDtypeStruct((N, HID), jnp.float32),
            jax.ShapeDtypeStruct((N, HID), jnp.float32),
        ],
    )(degp, h0, W1)


def _t23_body(part_ref, g_ref, dinv_ref, b_ref, w_ref, gn_ref):
    dinv = dinv_ref[...]
    h = dinv * (part_ref[0] + part_ref[1] + g_ref[...]) + b_ref[...]
    h = jnp.maximum(h, 0.0)
    gn_ref[...] = jnp.dot(h * dinv, w_ref[...],
                          preferred_element_type=jnp.float32)


def _t23(part, g, dinv, b, W):
    return pl.pallas_call(
        _t23_body,
        grid=(GR,),
        in_specs=[
            pl.BlockSpec((NC, RB, HID), lambda i: (0, i, 0)),
            pl.BlockSpec((RB, HID), lambda i: (i, 0)),
            pl.BlockSpec((RB, HID), lambda i: (i, 0)),
            pl.BlockSpec((1, HID), lambda i: (0, 0)),
            pl.BlockSpec((HID, HID), lambda i: (0, 0)),
        ],
        out_specs=pl.BlockSpec((RB, HID), lambda i: (i, 0)),
        out_shape=jax.ShapeDtypeStruct((N, HID), jnp.float32),
    )(part, g, dinv, b, W)


def _t4a_body(part_ref, g_ref, dinv_ref, b_ref, batch_ref,
              wlin_ref, blin_ref, wih0_ref, whh0_ref, bb0_ref,
              wih1_ref, whh1_ref, bb1_ref, o_ref, sum_scr, cnt_scr):
    i = pl.program_id(0)
    dinv = dinv_ref[...]
    h = dinv * (part_ref[0] + part_ref[1] + g_ref[...]) + b_ref[...]
    b_blk = batch_ref[0, 0, :]
    gids = lax.broadcasted_iota(jnp.int32, (NG, RB), 0)
    M = jnp.where(gids == jnp.broadcast_to(b_blk[None, :], (NG, RB)),
                  1.0, 0.0)
    psum = jnp.dot(M, h, preferred_element_type=jnp.float32)
    pcnt = jnp.broadcast_to(jnp.sum(M, axis=1, keepdims=True), (NG, HID))

    @pl.when(i == 0)
    def _():
        sum_scr[...] = psum
        cnt_scr[...] = pcnt

    @pl.when(i > 0)
    def _():
        sum_scr[...] += psum
        cnt_scr[...] += pcnt

    @pl.when(i == GR - 1)
    def _():
        pooled = sum_scr[...] / jnp.maximum(cnt_scr[...], 1.0)
        lin = jnp.dot(pooled, wlin_ref[...],
                      preferred_element_type=jnp.float32) + blin_ref[...]
        xw0 = jnp.dot(lin, wih0_ref[...],
                      preferred_element_type=jnp.float32) + bb0_ref[...]
        hh = jnp.zeros((NG, LS), jnp.float32)
        cc = jnp.zeros((NG, LS), jnp.float32)
        ys = []
        for t in range(SEQ):
            gts = xw0 + jnp.dot(hh, whh0_ref[...],
                                preferred_element_type=jnp.float32)
            ig = jax.nn.sigmoid(gts[:, 0:LS])
            fg = jax.nn.sigmoid(gts[:, LS:2 * LS])
            gg = jnp.tanh(gts[:, 2 * LS:3 * LS])
            og = jax.nn.sigmoid(gts[:, 3 * LS:4 * LS])
            cc = fg * cc + ig * gg
            hh = og * jnp.tanh(cc)
            ys.append(hh)
        hh1 = jnp.zeros((NG, LS), jnp.float32)
        cc1 = jnp.zeros((NG, LS), jnp.float32)
        for t in range(SEQ):
            gts = (jnp.dot(ys[t], wih1_ref[...],
                           preferred_element_type=jnp.float32)
                   + jnp.dot(hh1, whh1_ref[...],
                             preferred_element_type=jnp.float32)
                   + bb1_ref[...])
            ig = jax.nn.sigmoid(gts[:, 0:LS])
            fg = jax.nn.sigmoid(gts[:, LS:2 * LS])
            gg = jnp.tanh(gts[:, 2 * LS:3 * LS])
            og = jax.nn.sigmoid(gts[:, 3 * LS:4 * LS])
            cc1 = fg * cc1 + ig * gg
            hh1 = og * jnp.tanh(cc1)
            o_ref[t] = hh1


def _t4a(part, g, dinv, b, batch3, Wlin, blin, Wih0T, Whh0T, bb0,
         Wih1T, Whh1T, bb1):
    full = lambda shape: pl.BlockSpec(shape, lambda i: tuple(0 for _ in shape))
    return pl.pallas_call(
        _t4a_body,
        grid=(GR,),
        in_specs=[
            pl.BlockSpec((NC, RB, HID), lambda i: (0, i, 0)),
            pl.BlockSpec((RB, HID), lambda i: (i, 0)),
            pl.BlockSpec((RB, HID), lambda i: (i, 0)),
            full((1, HID)),
            pl.BlockSpec((1, 1, RB), lambda i: (i, 0, 0)),
            full((HID, HID)),
            full((1, HID)),
            full((HID, 4 * LS)),
            full((LS, 4 * LS)),
            full((1, 4 * LS)),
            full((LS, 4 * LS)),
            full((LS, 4 * LS)),
            full((1, 4 * LS)),
        ],
        out_specs=pl.BlockSpec((SEQ, NG, LS), lambda i: (0, 0, 0)),
        out_shape=jax.ShapeDtypeStruct((SEQ, NG, LS), jnp.float32),
        scratch_shapes=[
            pltpu.VMEM((NG, HID), jnp.float32),
            pltpu.VMEM((NG, HID), jnp.float32),
        ],
    )(part, g, dinv, b, batch3, Wlin, blin, Wih0T, Whh0T, bb0,
      Wih1T, Whh1T, bb1)


def _t4b_body(o_ref, wd_ref, bd_ref, out_ref):
    out_ref[...] = jnp.dot(o_ref[...], wd_ref[...],
                           preferred_element_type=jnp.float32) + bd_ref[...]


RB4 = 256


def _t4b(o2d, Wd, bd):
    return pl.pallas_call(
        _t4b_body,
        grid=(NG * SEQ // RB4,),
        in_specs=[
            pl.BlockSpec((RB4, LS), lambda j: (j, 0)),
            pl.BlockSpec((LS, NV), lambda j: (0, 0)),
            pl.BlockSpec((1, NV), lambda j: (0, 0)),
        ],
        out_specs=pl.BlockSpec((RB4, NV), lambda j: (j, 0)),
        out_shape=jax.ShapeDtypeStruct((NG * SEQ, NV), jnp.float32),
    )(o2d, Wd, bd)


def kernel(x, edge_index, edge_attr, batch, embed,
           W1, b1, W2, b2, W3, b3, Wlin, blin,
           Wih0, Whh0, bih0, bhh0, Wih1, Whh1, bih1, bhh1, Wd, bd):
    del edge_attr
    xidx = x[:, 0]
    src2 = edge_index[0].reshape(NW, NCH, KS)
    dst2 = edge_index[1].reshape(NW, NCH, 1, KS)
    dst2h = edge_index[1].reshape(NW, CPW, K)
    ones128 = jnp.ones((K, HID), jnp.float32)
    z128 = jnp.zeros((N, HID), jnp.float32)

    h0, degp = _embed_deg(embed, xidx, dst2h, ones128, z128)
    dinv, g1 = _t1(degp, h0, W1)
    p1 = _segsum(g1, src2, dst2, z128)
    g2 = _t23(p1, g1, dinv, b1.reshape(1, HID), W2)
    p2 = _segsum(g2, src2, dst2, z128)
    g3 = _t23(p2, g2, dinv, b2.reshape(1, HID), W3)
    p3 = _segsum(g3, src2, dst2, z128)

    batch3 = batch.reshape(GR, 1, RB)
    o = _t4a(p3, g3, dinv, b3.reshape(1, HID), batch3,
             Wlin, blin.reshape(1, HID),
             Wih0.T, Whh0.T, (bih0 + bhh0).reshape(1, 4 * LS),
             Wih1.T, Whh1.T, (bih1 + bhh1).reshape(1, 4 * LS))
    o2d = jnp.swapaxes(o, 0, 1).reshape(NG * SEQ, LS)
    logits = _t4b(o2d, Wd, bd.reshape(1, NV))
    return logits.reshape(NG, SEQ, NV)


# speedup vs baseline: 22.5948x; 1.0030x over previous
import functools

import jax
import jax.numpy as jnp
from jax import lax
from jax.experimental import pallas as pl
from jax.experimental.pallas import tpu as pltpu
from jax.experimental.pallas import tpu_sc as plsc

N = 10000
E = 320000
EMB = 128
HID = 128
NG = 64
SEQ = 20
LS = 128
NV = 5000

NC = 2
NS = 16
NW = NC * NS
K = 80
CPW = E // (NW * K)
KS = 80
NCH = E // (NW * KS)
NBUF = 3
EPW = E // NW
RPT = 624
TAIL_BASE = NS * RPT
TAIL = N - TAIL_BASE
ECH = N // K

RB = 1000
GR = N // RB

_MESH = plsc.VectorSubcoreMesh(core_axis_name="c", subcore_axis_name="s")


def _stripe_copy(sid, src, dst):
    pltpu.sync_copy(src.at[pl.ds(sid * RPT, RPT)],
                    dst.at[pl.ds(sid * RPT, RPT)])

    @pl.when(sid == NS - 1)
    def _():
        pltpu.sync_copy(src.at[pl.ds(TAIL_BASE, TAIL)],
                        dst.at[pl.ds(TAIL_BASE, TAIL)])


@functools.partial(
    pl.kernel,
    out_type=(
        jax.ShapeDtypeStruct((N, EMB), jnp.float32),
        jax.ShapeDtypeStruct((NC, N, HID), jnp.float32),
    ),
    mesh=_MESH,
    scratch_types=[
        pltpu.VMEM((K,), jnp.int32),
        pltpu.VMEM((K, EMB), jnp.float32),
        pltpu.VMEM((CPW, 1, K), jnp.int32),
        pltpu.VMEM((K, HID), jnp.float32),
        pltpu.SemaphoreType.DMA,
        pltpu.SemaphoreType.DMA,
        pltpu.VMEM_SHARED((N, HID), jnp.float32),
    ],
)
def _embed_deg(emb_hbm, xidx_hbm, dst_hbm, ones_hbm, z_hbm,
               h0_hbm, degp_hbm, xbuf, rows, didx, ones_v, sem, dsem, deg_sh):
    cid = lax.axis_index("c")
    sid = lax.axis_index("s")
    wid = sid * NC + cid

    _stripe_copy(sid, z_hbm, deg_sh)
    pltpu.sync_copy(ones_hbm, ones_v)
    pltpu.sync_copy(dst_hbm.at[wid], didx)

    for j in range((ECH + NW - 1) // NW):
        ch = wid + j * NW

        @pl.when(ch < ECH)
        def _():
            pltpu.sync_copy(xidx_hbm.at[pl.ds(ch * K, K)], xbuf)
            pltpu.async_copy(emb_hbm.at[xbuf], rows, sem).wait()
            pltpu.sync_copy(rows, h0_hbm.at[pl.ds(ch * K, K)])

    plsc.subcore_barrier()

    DD = 4

    def body(i, carry):
        @pl.when(i >= DD)
        def _():
            pltpu.make_async_copy(ones_v, deg_sh.at[didx.at[0].at[0]],
                                  dsem).wait()

        pltpu.async_copy(ones_v, deg_sh.at[didx.at[i].at[0]], dsem,
                         add=True)
        return carry

    lax.fori_loop(0, CPW, body, 0)
    for _t in range(DD):
        pltpu.make_async_copy(ones_v, deg_sh.at[didx.at[0].at[0]],
                              dsem).wait()
    plsc.subcore_barrier()
    _stripe_copy(sid, deg_sh, degp_hbm.at[cid])


@functools.partial(
    pl.kernel,
    out_type=jax.ShapeDtypeStruct((NC, N, HID), jnp.float32),
    mesh=_MESH,
    scratch_types=[
        pltpu.VMEM((NCH, KS), jnp.int32),
        [pltpu.VMEM((1, KS), jnp.int32) for _ in range(NBUF)],
        [pltpu.VMEM((KS, HID), jnp.float32) for _ in range(NBUF)],
        [pltpu.SemaphoreType.DMA for _ in range(NBUF)],
        [pltpu.SemaphoreType.DMA for _ in range(NBUF)],
        pltpu.VMEM_SHARED((N, HID), jnp.float32),
    ],
)
def _segsum(g_hbm, src_hbm, dst_hbm, z_hbm,
            part_hbm, sidx, didx, rows, gsems, dsems, acc_sh):
    cid = lax.axis_index("c")
    sid = lax.axis_index("s")
    wid = sid * NC + cid

    _stripe_copy(sid, z_hbm, acc_sh)
    pltpu.sync_copy(src_hbm.at[wid], sidx)

    def fetch(i, b):
        pltpu.async_copy(dst_hbm.at[wid, i], didx[b], dsems[b])
        pltpu.async_copy(g_hbm.at[sidx.at[i]], rows[b], gsems[b])

    def drain(i, b):
        pltpu.make_async_copy(dst_hbm.at[wid, i], didx[b],
                              dsems[b]).wait()
        pltpu.make_async_copy(g_hbm.at[sidx.at[i]], rows[b],
                              gsems[b]).wait()
        pltpu.sync_copy(rows[b], acc_sh.at[didx[b].at[0]], add=True)

    for b in range(NBUF):
        fetch(b, b)
    plsc.subcore_barrier()

    def body(j, carry):
        for b in range(NBUF):
            i = NBUF * j + b
            drain(i, b)
            nxt = i + NBUF

            @pl.when(nxt < NCH)
            def _():
                fetch(nxt, b)
        return carry

    lax.fori_loop(0, NCH // NBUF, body, 0)
    for i in range(NBUF * (NCH // NBUF), NCH):
        drain(i, i % NBUF)
    plsc.subcore_barrier()
    _stripe_copy(sid, acc_sh, part_hbm.at[cid])


def _t1_body(degp_ref, h0_ref, w_ref, dinv_ref, g_ref):
    d = degp_ref[0] + degp_ref[1]
    deg = d[:, 0:1] + 1.0
    dinv = jnp.broadcast_to(lax.rsqrt(deg), (RB, HID))
    dinv_ref[...] = dinv
    g_ref[...] = jnp.dot(h0_ref[...] * dinv, w_ref[...],
                         preferred_element_type=jnp.float32)


def _t1(degp, h0, W1):
    return pl.pallas_call(
        _t1_body,
        grid=(GR,),
        in_specs=[
            pl.BlockSpec((NC, RB, HID), lambda i: (0, i, 0)),
            pl.BlockSpec((RB, EMB), lambda i: (i, 0)),
            pl.BlockSpec((EMB, HID), lambda i: (0, 0)),
        ],
        out_specs=[
            pl.BlockSpec((RB, HID), lambda i: (i, 0)),
            pl.BlockSpec((RB, HID), lambda i: (i, 0)),
        ],
        out_shape=[
            jax.ShapeDtypeStruct((N, HID), jnp.float32),
            jax.ShapeDtypeStruct((N, HID), jnp.float32),
        ],
    )(degp, h0, W1)


def _t23_body(part_ref, g_ref, dinv_ref, b_ref, w_ref, gn_ref):
    dinv = dinv_ref[...]
    h = dinv * (part_ref[0] + part_ref[1] + g_ref[...]) + b_ref[...]
    h = jnp.maximum(h, 0.0)
    gn_ref[...] = jnp.dot(h * dinv, w_ref[...],
                          preferred_element_type=jnp.float32)


def _t23(part, g, dinv, b, W):
    return pl.pallas_call(
        _t23_body,
        grid=(GR,),
        in_specs=[
            pl.BlockSpec((NC, RB, HID), lambda i: (0, i, 0)),
            pl.BlockSpec((RB, HID), lambda i: (i, 0)),
            pl.BlockSpec((RB, HID), lambda i: (i, 0)),
            pl.BlockSpec((1, HID), lambda i: (0, 0)),
            pl.BlockSpec((HID, HID), lambda i: (0, 0)),
        ],
        out_specs=pl.BlockSpec((RB, HID), lambda i: (i, 0)),
        out_shape=jax.ShapeDtypeStruct((N, HID), jnp.float32),
    )(part, g, dinv, b, W)


def _t4a_body(part_ref, g_ref, dinv_ref, b_ref, batch_ref,
              wlin_ref, blin_ref, wih0_ref, whh0_ref, bb0_ref,
              wih1_ref, whh1_ref, bb1_ref, o_ref, sum_scr, cnt_scr):
    i = pl.program_id(0)
    dinv = dinv_ref[...]
    h = dinv * (part_ref[0] + part_ref[1] + g_ref[...]) + b_ref[...]
    b_blk = batch_ref[0, 0, :]
    gids = lax.broadcasted_iota(jnp.int32, (NG, RB), 0)
    M = jnp.where(gids == jnp.broadcast_to(b_blk[None, :], (NG, RB)),
                  1.0, 0.0)
    psum = jnp.dot(M, h, preferred_element_type=jnp.float32)
    pcnt = jnp.broadcast_to(jnp.sum(M, axis=1, keepdims=True), (NG, HID))

    @pl.when(i == 0)
    def _():
        sum_scr[...] = psum
        cnt_scr[...] = pcnt

    @pl.when(i > 0)
    def _():
        sum_scr[...] += psum
        cnt_scr[...] += pcnt

    @pl.when(i == GR - 1)
    def _():
        pooled = sum_scr[...] / jnp.maximum(cnt_scr[...], 1.0)
        lin = jnp.dot(pooled, wlin_ref[...],
                      preferred_element_type=jnp.float32) + blin_ref[...]
        xw0 = jnp.dot(lin, wih0_ref[...],
                      preferred_element_type=jnp.float32) + bb0_ref[...]
        hh = jnp.zeros((NG, LS), jnp.float32)
        cc = jnp.zeros((NG, LS), jnp.float32)
        ys = []
        for t in range(SEQ):
            gts = xw0 + jnp.dot(hh, whh0_ref[...],
                                preferred_element_type=jnp.float32)
            ig = jax.nn.sigmoid(gts[:, 0:LS])
            fg = jax.nn.sigmoid(gts[:, LS:2 * LS])
            gg = jnp.tanh(gts[:, 2 * LS:3 * LS])
            og = jax.nn.sigmoid(gts[:, 3 * LS:4 * LS])
            cc = fg * cc + ig * gg
            hh = og * jnp.tanh(cc)
            ys.append(hh)
        hh1 = jnp.zeros((NG, LS), jnp.float32)
        cc1 = jnp.zeros((NG, LS), jnp.float32)
        for t in range(SEQ):
            gts = (jnp.dot(ys[t], wih1_ref[...],
                           preferred_element_type=jnp.float32)
                   + jnp.dot(hh1, whh1_ref[...],
                             preferred_element_type=jnp.float32)
                   + bb1_ref[...])
            ig = jax.nn.sigmoid(gts[:, 0:LS])
            fg = jax.nn.sigmoid(gts[:, LS:2 * LS])
            gg = jnp.tanh(gts[:, 2 * LS:3 * LS])
            og = jax.nn.sigmoid(gts[:, 3 * LS:4 * LS])
            cc1 = fg * cc1 + ig * gg
            hh1 = og * jnp.tanh(cc1)
            o_ref[t] = hh1


def _t4a(part, g, dinv, b, batch3, Wlin, blin, Wih0T, Whh0T, bb0,
         Wih1T, Whh1T, bb1):
    full = lambda shape: pl.BlockSpec(shape, lambda i: tuple(0 for _ in shape))
    return pl.pallas_call(
        _t4a_body,
        grid=(GR,),
        in_specs=[
            pl.BlockSpec((NC, RB, HID), lambda i: (0, i, 0)),
            pl.BlockSpec((RB, HID), lambda i: (i, 0)),
            pl.BlockSpec((RB, HID), lambda i: (i, 0)),
            full((1, HID)),
            pl.BlockSpec((1, 1, RB), lambda i: (i, 0, 0)),
            full((HID, HID)),
            full((1, HID)),
            full((HID, 4 * LS)),
            full((LS, 4 * LS)),
            full((1, 4 * LS)),
            full((LS, 4 * LS)),
            full((LS, 4 * LS)),
            full((1, 4 * LS)),
        ],
        out_specs=pl.BlockSpec((SEQ, NG, LS), lambda i: (0, 0, 0)),
        out_shape=jax.ShapeDtypeStruct((SEQ, NG, LS), jnp.float32),
        scratch_shapes=[
            pltpu.VMEM((NG, HID), jnp.float32),
            pltpu.VMEM((NG, HID), jnp.float32),
        ],
    )(part, g, dinv, b, batch3, Wlin, blin, Wih0T, Whh0T, bb0,
      Wih1T, Whh1T, bb1)


def _t4b_body(o_ref, wd_ref, bd_ref, out_ref):
    out_ref[...] = jnp.dot(o_ref[...], wd_ref[...],
                           preferred_element_type=jnp.float32) + bd_ref[...]


RB4 = 256


def _t4b(o2d, Wd, bd):
    return pl.pallas_call(
        _t4b_body,
        grid=(NG * SEQ // RB4,),
        in_specs=[
            pl.BlockSpec((RB4, LS), lambda j: (j, 0)),
            pl.BlockSpec((LS, NV), lambda j: (0, 0)),
            pl.BlockSpec((1, NV), lambda j: (0, 0)),
        ],
        out_specs=pl.BlockSpec((RB4, NV), lambda j: (j, 0)),
        out_shape=jax.ShapeDtypeStruct((NG * SEQ, NV), jnp.float32),
    )(o2d, Wd, bd)


def kernel(x, edge_index, edge_attr, batch, embed,
           W1, b1, W2, b2, W3, b3, Wlin, blin,
           Wih0, Whh0, bih0, bhh0, Wih1, Whh1, bih1, bhh1, Wd, bd):
    del edge_attr
    xidx = x[:, 0]
    src2 = edge_index[0].reshape(NW, NCH, KS)
    dst2 = edge_index[1].reshape(NW, NCH, 1, KS)
    ones128 = jnp.ones((K, HID), jnp.float32)
    z128 = jnp.zeros((N, HID), jnp.float32)

    h0, degp = _embed_deg(embed, xidx, dst2, ones128, z128)
    dinv, g1 = _t1(degp, h0, W1)
    p1 = _segsum(g1, src2, dst2, z128)
    g2 = _t23(p1, g1, dinv, b1.reshape(1, HID), W2)
    p2 = _segsum(g2, src2, dst2, z128)
    g3 = _t23(p2, g2, dinv, b2.reshape(1, HID), W3)
    p3 = _segsum(g3, src2, dst2, z128)

    batch3 = batch.reshape(GR, 1, RB)
    o = _t4a(p3, g3, dinv, b3.reshape(1, HID), batch3,
             Wlin, blin.reshape(1, HID),
             Wih0.T, Whh0.T, (bih0 + bhh0).reshape(1, 4 * LS),
             Wih1.T, Whh1.T, (bih1 + bhh1).reshape(1, 4 * LS))
    o2d = jnp.swapaxes(o, 0, 1).reshape(NG * SEQ, LS)
    logits = _t4b(o2d, Wd, bd.reshape(1, NV))
    return logits.reshape(NG, SEQ, NV)


# speedup vs baseline: 22.7376x; 1.0063x over previous
import functools

import jax
import jax.numpy as jnp
from jax import lax
from jax.experimental import pallas as pl
from jax.experimental.pallas import tpu as pltpu
from jax.experimental.pallas import tpu_sc as plsc

N = 10000
E = 320000
EMB = 128
HID = 128
NG = 64
SEQ = 20
LS = 128
NV = 5000

NC = 2
NS = 16
NW = NC * NS
K = 80
CPW = E // (NW * K)
KS = 80
NCH = E // (NW * KS)
NBUF = 3
EPW = E // NW
RPT = 624
TAIL_BASE = NS * RPT
TAIL = N - TAIL_BASE
ECH = N // K

RB = 1000
GR = N // RB

_MESH = plsc.VectorSubcoreMesh(core_axis_name="c", subcore_axis_name="s")


def _stripe_copy(sid, src, dst):
    pltpu.sync_copy(src.at[pl.ds(sid * RPT, RPT)],
                    dst.at[pl.ds(sid * RPT, RPT)])

    @pl.when(sid == NS - 1)
    def _():
        pltpu.sync_copy(src.at[pl.ds(TAIL_BASE, TAIL)],
                        dst.at[pl.ds(TAIL_BASE, TAIL)])


@functools.partial(
    pl.kernel,
    out_type=(
        jax.ShapeDtypeStruct((N, EMB), jnp.float32),
        jax.ShapeDtypeStruct((NC, N, HID), jnp.float32),
    ),
    mesh=_MESH,
    scratch_types=[
        pltpu.VMEM((K,), jnp.int32),
        pltpu.VMEM((K, EMB), jnp.float32),
        [pltpu.VMEM((K,), jnp.int32) for _ in range(4)],
        pltpu.VMEM((K, HID), jnp.float32),
        pltpu.SemaphoreType.DMA,
        [pltpu.SemaphoreType.DMA for _ in range(4)],
        pltpu.VMEM_SHARED((N, HID), jnp.float32),
    ],
)
def _embed_deg(emb_hbm, xidx_hbm, dst_hbm, ones_hbm, z_hbm,
               h0_hbm, degp_hbm, xbuf, rows, didx, ones_v, sem, dsems, deg_sh):
    cid = lax.axis_index("c")
    sid = lax.axis_index("s")
    wid = sid * NC + cid

    _stripe_copy(sid, z_hbm, deg_sh)
    pltpu.sync_copy(ones_hbm, ones_v)

    def dfetch(i, b):
        pltpu.async_copy(dst_hbm.at[pl.ds(wid * EPW + i * K, K)],
                         didx[b], dsems[b])

    for b in range(4):
        dfetch(b, b)

    for j in range((ECH + NW - 1) // NW):
        ch = wid + j * NW

        @pl.when(ch < ECH)
        def _():
            pltpu.sync_copy(xidx_hbm.at[pl.ds(ch * K, K)], xbuf)
            pltpu.async_copy(emb_hbm.at[xbuf], rows, sem).wait()
            pltpu.sync_copy(rows, h0_hbm.at[pl.ds(ch * K, K)])

    plsc.subcore_barrier()

    def body(j, carry):
        for b in range(4):
            i = 4 * j + b
            pltpu.make_async_copy(dst_hbm.at[pl.ds(wid * EPW + i * K, K)],
                                  didx[b], dsems[b]).wait()
            pltpu.sync_copy(ones_v, deg_sh.at[didx[b]], add=True)
            nxt = i + 4

            @pl.when(nxt < CPW)
            def _():
                dfetch(nxt, b)
        return carry

    lax.fori_loop(0, CPW // 4, body, 0)
    for i in range(4 * (CPW // 4), CPW):
        b = i % 4
        pltpu.make_async_copy(dst_hbm.at[pl.ds(wid * EPW + i * K, K)],
                              didx[b], dsems[b]).wait()
        pltpu.sync_copy(ones_v, deg_sh.at[didx[b]], add=True)
    plsc.subcore_barrier()
    _stripe_copy(sid, deg_sh, degp_hbm.at[cid])


@functools.partial(
    pl.kernel,
    out_type=jax.ShapeDtypeStruct((NC, N, HID), jnp.float32),
    mesh=_MESH,
    scratch_types=[
        pltpu.VMEM((EPW,), jnp.int32),
        [pltpu.VMEM((KS,), jnp.int32) for _ in range(NBUF)],
        [pltpu.VMEM((KS, HID), jnp.float32) for _ in range(NBUF)],
        [pltpu.SemaphoreType.DMA for _ in range(NBUF)],
        [pltpu.SemaphoreType.DMA for _ in range(NBUF)],
        pltpu.VMEM_SHARED((N, HID), jnp.float32),
    ],
)
def _segsum(g_hbm, src_hbm, dst_hbm, z_hbm,
            part_hbm, sidx, didx, rows, gsems, dsems, acc_sh):
    cid = lax.axis_index("c")
    sid = lax.axis_index("s")
    wid = sid * NC + cid

    _stripe_copy(sid, z_hbm, acc_sh)
    pltpu.sync_copy(src_hbm.at[pl.ds(wid * EPW, EPW)], sidx)

    def fetch(i, b):
        pltpu.async_copy(dst_hbm.at[pl.ds(wid * EPW + i * KS, KS)],
                         didx[b], dsems[b])
        pltpu.async_copy(g_hbm.at[sidx.at[pl.ds(i * KS, KS)]],
                         rows[b], gsems[b])

    def drain(i, b):
        pltpu.make_async_copy(dst_hbm.at[pl.ds(wid * EPW + i * KS, KS)],
                              didx[b], dsems[b]).wait()
        pltpu.make_async_copy(g_hbm.at[sidx.at[pl.ds(i * KS, KS)]],
                              rows[b], gsems[b]).wait()
        pltpu.sync_copy(rows[b], acc_sh.at[didx[b]], add=True)

    for b in range(NBUF):
        fetch(b, b)
    plsc.subcore_barrier()

    def body(j, carry):
        for b in range(NBUF):
            i = NBUF * j + b
            drain(i, b)
            nxt = i + NBUF

            @pl.when(nxt < NCH)
            def _():
                fetch(nxt, b)
        return carry

    lax.fori_loop(0, NCH // NBUF, body, 0)
    for i in range(NBUF * (NCH // NBUF), NCH):
        drain(i, i % NBUF)
    plsc.subcore_barrier()
    _stripe_copy(sid, acc_sh, part_hbm.at[cid])


def _t1_body(degp_ref, h0_ref, w_ref, dinv_ref, g_ref):
    d = degp_ref[0] + degp_ref[1]
    deg = d[:, 0:1] + 1.0
    dinv = jnp.broadcast_to(lax.rsqrt(deg), (RB, HID))
    dinv_ref[...] = dinv
    g_ref[...] = jnp.dot(h0_ref[...] * dinv, w_ref[...],
                         preferred_element_type=jnp.float32)


def _t1(degp, h0, W1):
    return pl.pallas_call(
        _t1_body,
        grid=(GR,),
        in_specs=[
            pl.BlockSpec((NC, RB, HID), lambda i: (0, i, 0)),
            pl.BlockSpec((RB, EMB), lambda i: (i, 0)),
            pl.BlockSpec((EMB, HID), lambda i: (0, 0)),
        ],
        out_specs=[
            pl.BlockSpec((RB, HID), lambda i: (i, 0)),
            pl.BlockSpec((RB, HID), lambda i: (i, 0)),
        ],
        out_shape=[
            jax.ShapeDtypeStruct((N, HID), jnp.float32),
            jax.ShapeDtypeStruct((N, HID), jnp.float32),
        ],
    )(degp, h0, W1)


def _t23_body(part_ref, g_ref, dinv_ref, b_ref, w_ref, gn_ref):
    dinv = dinv_ref[...]
    h = dinv * (part_ref[0] + part_ref[1] + g_ref[...]) + b_ref[...]
    h = jnp.maximum(h, 0.0)
    gn_ref[...] = jnp.dot(h * dinv, w_ref[...],
                          preferred_element_type=jnp.float32)


def _t23(part, g, dinv, b, W):
    return pl.pallas_call(
        _t23_body,
        grid=(GR,),
        in_specs=[
            pl.BlockSpec((NC, RB, HID), lambda i: (0, i, 0)),
            pl.BlockSpec((RB, HID), lambda i: (i, 0)),
            pl.BlockSpec((RB, HID), lambda i: (i, 0)),
            pl.BlockSpec((1, HID), lambda i: (0, 0)),
            pl.BlockSpec((HID, HID), lambda i: (0, 0)),
        ],
        out_specs=pl.BlockSpec((RB, HID), lambda i: (i, 0)),
        out_shape=jax.ShapeDtypeStruct((N, HID), jnp.float32),
    )(part, g, dinv, b, W)


def _t4a_body(part_ref, g_ref, dinv_ref, b_ref, batch_ref,
              wlin_ref, blin_ref, wih0_ref, whh0_ref, bb0_ref,
              wih1_ref, whh1_ref, bb1_ref, o_ref, sum_scr, cnt_scr):
    i = pl.program_id(0)
    dinv = dinv_ref[...]
    h = dinv * (part_ref[0] + part_ref[1] + g_ref[...]) + b_ref[...]
    b_blk = batch_ref[0, 0, :]
    gids = lax.broadcasted_iota(jnp.int32, (NG, RB), 0)
    M = jnp.where(gids == jnp.broadcast_to(b_blk[None, :], (NG, RB)),
                  1.0, 0.0)
    psum = jnp.dot(M, h, preferred_element_type=jnp.float32)
    pcnt = jnp.broadcast_to(jnp.sum(M, axis=1, keepdims=True), (NG, HID))

    @pl.when(i == 0)
    def _():
        sum_scr[...] = psum
        cnt_scr[...] = pcnt

    @pl.when(i > 0)
    def _():
        sum_scr[...] += psum
        cnt_scr[...] += pcnt

    @pl.when(i == GR - 1)
    def _():
        pooled = sum_scr[...] / jnp.maximum(cnt_scr[...], 1.0)
        lin = jnp.dot(pooled, wlin_ref[...],
                      preferred_element_type=jnp.float32) + blin_ref[...]
        xw0 = jnp.dot(lin, wih0_ref[...],
                      preferred_element_type=jnp.float32) + bb0_ref[...]
        hh = jnp.zeros((NG, LS), jnp.float32)
        cc = jnp.zeros((NG, LS), jnp.float32)
        ys = []
        for t in range(SEQ):
            gts = xw0 + jnp.dot(hh, whh0_ref[...],
                                preferred_element_type=jnp.float32)
            ig = jax.nn.sigmoid(gts[:, 0:LS])
            fg = jax.nn.sigmoid(gts[:, LS:2 * LS])
            gg = jnp.tanh(gts[:, 2 * LS:3 * LS])
            og = jax.nn.sigmoid(gts[:, 3 * LS:4 * LS])
            cc = fg * cc + ig * gg
            hh = og * jnp.tanh(cc)
            ys.append(hh)
        hh1 = jnp.zeros((NG, LS), jnp.float32)
        cc1 = jnp.zeros((NG, LS), jnp.float32)
        for t in range(SEQ):
            gts = (jnp.dot(ys[t], wih1_ref[...],
                           preferred_element_type=jnp.float32)
                   + jnp.dot(hh1, whh1_ref[...],
                             preferred_element_type=jnp.float32)
                   + bb1_ref[...])
            ig = jax.nn.sigmoid(gts[:, 0:LS])
            fg = jax.nn.sigmoid(gts[:, LS:2 * LS])
            gg = jnp.tanh(gts[:, 2 * LS:3 * LS])
            og = jax.nn.sigmoid(gts[:, 3 * LS:4 * LS])
            cc1 = fg * cc1 + ig * gg
            hh1 = og * jnp.tanh(cc1)
            o_ref[t] = hh1


def _t4a(part, g, dinv, b, batch3, Wlin, blin, Wih0T, Whh0T, bb0,
         Wih1T, Whh1T, bb1):
    full = lambda shape: pl.BlockSpec(shape, lambda i: tuple(0 for _ in shape))
    return pl.pallas_call(
        _t4a_body,
        grid=(GR,),
        in_specs=[
            pl.BlockSpec((NC, RB, HID), lambda i: (0, i, 0)),
            pl.BlockSpec((RB, HID), lambda i: (i, 0)),
            pl.BlockSpec((RB, HID), lambda i: (i, 0)),
            full((1, HID)),
            pl.BlockSpec((1, 1, RB), lambda i: (i, 0, 0)),
            full((HID, HID)),
            full((1, HID)),
            full((HID, 4 * LS)),
            full((LS, 4 * LS)),
            full((1, 4 * LS)),
            full((LS, 4 * LS)),
            full((LS, 4 * LS)),
            full((1, 4 * LS)),
        ],
        out_specs=pl.BlockSpec((SEQ, NG, LS), lambda i: (0, 0, 0)),
        out_shape=jax.ShapeDtypeStruct((SEQ, NG, LS), jnp.float32),
        scratch_shapes=[
            pltpu.VMEM((NG, HID), jnp.float32),
            pltpu.VMEM((NG, HID), jnp.float32),
        ],
    )(part, g, dinv, b, batch3, Wlin, blin, Wih0T, Whh0T, bb0,
      Wih1T, Whh1T, bb1)


def _t4b_body(o_ref, wd_ref, bd_ref, out_ref):
    out_ref[...] = jnp.dot(o_ref[...], wd_ref[...],
                           preferred_element_type=jnp.float32) + bd_ref[...]


RB4 = 256


def _t4b(o2d, Wd, bd):
    return pl.pallas_call(
        _t4b_body,
        grid=(NG * SEQ // RB4,),
        in_specs=[
            pl.BlockSpec((RB4, LS), lambda j: (j, 0)),
            pl.BlockSpec((LS, NV), lambda j: (0, 0)),
            pl.BlockSpec((1, NV), lambda j: (0, 0)),
        ],
        out_specs=pl.BlockSpec((RB4, NV), lambda j: (j, 0)),
        out_shape=jax.ShapeDtypeStruct((NG * SEQ, NV), jnp.float32),
    )(o2d, Wd, bd)


def kernel(x, edge_index, edge_attr, batch, embed,
           W1, b1, W2, b2, W3, b3, Wlin, blin,
           Wih0, Whh0, bih0, bhh0, Wih1, Whh1, bih1, bhh1, Wd, bd):
    del edge_attr
    xidx = x[:, 0]
    src1 = edge_index[0]
    dst1 = edge_index[1]
    ones128 = jnp.ones((K, HID), jnp.float32)
    z128 = jnp.zeros((N, HID), jnp.float32)

    h0, degp = _embed_deg(embed, xidx, dst1, ones128, z128)
    dinv, g1 = _t1(degp, h0, W1)
    p1 = _segsum(g1, src1, dst1, z128)
    g2 = _t23(p1, g1, dinv, b1.reshape(1, HID), W2)
    p2 = _segsum(g2, src1, dst1, z128)
    g3 = _t23(p2, g2, dinv, b2.reshape(1, HID), W3)
    p3 = _segsum(g3, src1, dst1, z128)

    batch3 = batch.reshape(GR, 1, RB)
    o = _t4a(p3, g3, dinv, b3.reshape(1, HID), batch3,
             Wlin, blin.reshape(1, HID),
             Wih0.T, Whh0.T, (bih0 + bhh0).reshape(1, 4 * LS),
             Wih1.T, Whh1.T, (bih1 + bhh1).reshape(1, 4 * LS))
    o2d = jnp.swapaxes(o, 0, 1).reshape(NG * SEQ, LS)
    logits = _t4b(o2d, Wd, bd.reshape(1, NV))
    return logits.reshape(NG, SEQ, NV)
